# baseline (speedup 1.0000x reference)
import functools

import jax
import jax.numpy as jnp
from jax.experimental import pallas as pl
from jax.experimental.pallas import tpu as pltpu

_BLOCK_T = 512


def _router_kernel(x_ref, w_ref, idx_ref, wgt_ref, cnt_ref, psum_ref,
                   lb_ref, z_ref, *, num_tokens, num_experts, num_blocks):
    i = pl.program_id(0)

    x = x_ref[...]
    w = w_ref[...]
    logits = jnp.dot(x, w, preferred_element_type=jnp.float32)

    m = jnp.max(logits, axis=-1, keepdims=True)
    ex = jnp.exp(logits - m)
    se = jnp.sum(ex, axis=-1, keepdims=True)
    inv_se = 1.0 / se
    probs = ex * inv_se

    idx = jnp.argmax(logits, axis=-1).astype(jnp.int32)
    idx_ref[...] = idx.reshape(1, 1, _BLOCK_T)
    wgt_ref[...] = inv_se.reshape(1, 1, _BLOCK_T)

    iota = jax.lax.broadcasted_iota(jnp.int32, (_BLOCK_T, num_experts), 1)
    part_cnt = jnp.sum((idx[:, None] == iota).astype(jnp.float32),
                       axis=0, keepdims=True)
    part_psum = jnp.sum(probs, axis=0, keepdims=True)
    lse = m + jnp.log(se)
    part_z = jnp.sum(lse * lse).reshape(1, 1)

    @pl.when(i == 0)
    def _init():
        cnt_ref[...] = part_cnt
        psum_ref[...] = part_psum
        z_ref[...] = part_z

    @pl.when(i > 0)
    def _acc():
        cnt_ref[...] += part_cnt
        psum_ref[...] += part_psum
        z_ref[...] += part_z

    @pl.when(i == num_blocks - 1)
    def _final():
        inv_n = 1.0 / num_tokens
        frac = cnt_ref[...] * inv_n
        meanp = psum_ref[...] * inv_n
        lb_ref[...] = (num_experts * jnp.sum(frac * meanp)).reshape(1, 1)
        psum_ref[...] = meanp
        z_ref[...] = z_ref[...] * inv_n


def kernel(hidden_states, gate_W):
    b, s, h = hidden_states.shape
    e = gate_W.shape[0]
    n = b * s
    x = hidden_states.reshape(n, h)
    wt = gate_W.T

    num_blocks = n // _BLOCK_T

    body = functools.partial(_router_kernel, num_tokens=n, num_experts=e,
                             num_blocks=num_blocks)
    row_spec = pl.BlockSpec((1, 1, _BLOCK_T), lambda i: (i, 0, 0))
    acc_spec = lambda shape: pl.BlockSpec(shape, lambda i: (0, 0))
    out_shapes = (
        jax.ShapeDtypeStruct((num_blocks, 1, _BLOCK_T), jnp.int32),
        jax.ShapeDtypeStruct((num_blocks, 1, _BLOCK_T), jnp.float32),
        jax.ShapeDtypeStruct((1, e), jnp.float32),
        jax.ShapeDtypeStruct((1, e), jnp.float32),
        jax.ShapeDtypeStruct((1, 1), jnp.float32),
        jax.ShapeDtypeStruct((1, 1), jnp.float32))
    out = pl.pallas_call(
        body,
        grid=(num_blocks,),
        in_specs=[
            pl.BlockSpec((_BLOCK_T, h), lambda i: (i, 0)),
            pl.BlockSpec((h, e), lambda i: (0, 0)),
        ],
        out_specs=(row_spec, row_spec,
                   acc_spec((1, e)), acc_spec((1, e)),
                   acc_spec((1, 1)), acc_spec((1, 1))),
        out_shape=out_shapes,
        compiler_params=pltpu.CompilerParams(
            vmem_limit_bytes=62 * 1024 * 1024),
    )(x, wt)

    idx, wgt, cnt, meanp, lb, z = out
    return (idx.reshape(b, s, 1), wgt.reshape(b, s, 1),
            lb.reshape(()), z.reshape(()),
            cnt.reshape(e), meanp.reshape(e))


# speedup vs baseline: 1.0823x; 1.0823x over previous
import functools

import jax
import jax.numpy as jnp
from jax.experimental import pallas as pl
from jax.experimental.pallas import tpu as pltpu

_BLOCK_T = 1024


def _router_kernel(x_ref, w_ref, idx_ref, wgt_ref, cnt_ref, psum_ref,
                   lb_ref, z_ref, *, num_tokens, num_experts, num_blocks):
    i = pl.program_id(0)

    x = x_ref[...]
    w = w_ref[...]
    logits = jnp.dot(x, w, preferred_element_type=jnp.float32)

    m = jnp.max(logits, axis=-1, keepdims=True)
    ex = jnp.exp(logits - m)
    se = jnp.sum(ex, axis=-1, keepdims=True)
    inv_se = 1.0 / se
    probs = ex * inv_se

    idx = jnp.argmax(logits, axis=-1).astype(jnp.int32)
    idx_ref[...] = idx.reshape(1, 1, _BLOCK_T)
    wgt_ref[...] = inv_se.reshape(1, 1, _BLOCK_T)

    iota = jax.lax.broadcasted_iota(jnp.int32, (_BLOCK_T, num_experts), 1)
    part_cnt = jnp.sum((idx[:, None] == iota).astype(jnp.float32),
                       axis=0, keepdims=True)
    part_psum = jnp.sum(probs, axis=0, keepdims=True)
    lse = m + jnp.log(se)
    part_z = jnp.sum(lse * lse).reshape(1, 1)

    @pl.when(i == 0)
    def _init():
        cnt_ref[...] = part_cnt
        psum_ref[...] = part_psum
        z_ref[...] = part_z

    @pl.when(i > 0)
    def _acc():
        cnt_ref[...] += part_cnt
        psum_ref[...] += part_psum
        z_ref[...] += part_z

    @pl.when(i == num_blocks - 1)
    def _final():
        inv_n = 1.0 / num_tokens
        frac = cnt_ref[...] * inv_n
        meanp = psum_ref[...] * inv_n
        lb_ref[...] = (num_experts * jnp.sum(frac * meanp)).reshape(1, 1)
        psum_ref[...] = meanp
        z_ref[...] = z_ref[...] * inv_n


def kernel(hidden_states, gate_W):
    b, s, h = hidden_states.shape
    e = gate_W.shape[0]
    n = b * s
    x = hidden_states.reshape(n, h)
    wt = gate_W.T

    num_blocks = n // _BLOCK_T

    body = functools.partial(_router_kernel, num_tokens=n, num_experts=e,
                             num_blocks=num_blocks)
    row_spec = pl.BlockSpec((1, 1, _BLOCK_T), lambda i: (i, 0, 0))
    acc_spec = lambda shape: pl.BlockSpec(shape, lambda i: (0, 0))
    out_shapes = (
        jax.ShapeDtypeStruct((num_blocks, 1, _BLOCK_T), jnp.int32),
        jax.ShapeDtypeStruct((num_blocks, 1, _BLOCK_T), jnp.float32),
        jax.ShapeDtypeStruct((1, e), jnp.float32),
        jax.ShapeDtypeStruct((1, e), jnp.float32),
        jax.ShapeDtypeStruct((1, 1), jnp.float32),
        jax.ShapeDtypeStruct((1, 1), jnp.float32))
    out = pl.pallas_call(
        body,
        grid=(num_blocks,),
        in_specs=[
            pl.BlockSpec((_BLOCK_T, h), lambda i: (i, 0)),
            pl.BlockSpec((h, e), lambda i: (0, 0)),
        ],
        out_specs=(row_spec, row_spec,
                   acc_spec((1, e)), acc_spec((1, e)),
                   acc_spec((1, 1)), acc_spec((1, 1))),
        out_shape=out_shapes,
        compiler_params=pltpu.CompilerParams(
            vmem_limit_bytes=62 * 1024 * 1024),
    )(x, wt)

    idx, wgt, cnt, meanp, lb, z = out
    return (idx.reshape(b, s, 1), wgt.reshape(b, s, 1),
            lb.reshape(()), z.reshape(()),
            cnt.reshape(e), meanp.reshape(e))
